# Initial kernel scaffold; baseline (speedup 1.0000x reference)
import jax
import jax.numpy as jnp
from jax.experimental import pallas as pl


def kernel(visit_ids, table):
    raise NotImplementedError("write your pallas kernel here")


# speedup vs baseline: 3.6220x; 3.6220x over previous
import functools

import jax
import jax.numpy as jnp
from jax import lax
from jax.experimental import pallas as pl
from jax.experimental.pallas import tpu as pltpu
from jax.experimental.pallas import tpu_sc as plsc

MAX_VISITS = 512
EMBED_DIM = 64
BATCH = 16384
SEQ = 200
N = BATCH * SEQ

_info = plsc.get_sparse_core_info()
NC, NS = _info.num_cores, _info.num_subcores
NW = NC * NS
PER_W = N // NW
CHUNK = 512
N_CHUNKS = PER_W // CHUNK

_mesh = plsc.VectorSubcoreMesh(core_axis_name="c", subcore_axis_name="s")


@functools.partial(
    pl.kernel,
    mesh=_mesh,
    out_type=jax.ShapeDtypeStruct((N, EMBED_DIM), jnp.float32),
    scratch_types=[
        pltpu.VMEM((CHUNK,), jnp.int32),
        pltpu.VMEM((CHUNK, EMBED_DIM), jnp.float32),
        pltpu.SemaphoreType.DMA,
    ],
    compiler_params=pltpu.CompilerParams(use_tc_tiling_on_sc=False),
)
def _gather_kernel(ids_hbm, table_hbm, out_hbm, idx_v, rows_v, sem):
    wid = lax.axis_index("s") * NC + lax.axis_index("c")

    def body(g, _):
        base = wid * PER_W + g * CHUNK
        pltpu.sync_copy(ids_hbm.at[pl.ds(base, CHUNK)], idx_v)
        pltpu.async_copy(table_hbm.at[idx_v], rows_v, sem).wait()
        pltpu.sync_copy(rows_v, out_hbm.at[pl.ds(base, CHUNK)])
        return ()

    lax.fori_loop(0, N_CHUNKS, body, ())


def kernel(visit_ids, table):
    ids_flat = visit_ids.reshape(N).astype(jnp.int32)
    out = _gather_kernel(ids_flat, table)
    return out.reshape(BATCH, SEQ, EMBED_DIM)


# speedup vs baseline: 7.5208x; 2.0764x over previous
import functools

import jax
import jax.numpy as jnp
from jax import lax
from jax.experimental import pallas as pl
from jax.experimental.pallas import tpu as pltpu
from jax.experimental.pallas import tpu_sc as plsc

MAX_VISITS = 512
EMBED_DIM = 64
BATCH = 16384
SEQ = 200

_info = plsc.get_sparse_core_info()
NC, NS = _info.num_cores, _info.num_subcores
NW = NC * NS
B_PER_W = BATCH // NW
NJ = B_PER_W // 16

_mesh = plsc.VectorSubcoreMesh(core_axis_name="c", subcore_axis_name="s")


@functools.partial(
    pl.kernel,
    mesh=_mesh,
    out_type=jax.ShapeDtypeStruct((SEQ, EMBED_DIM, BATCH), jnp.float32),
    scratch_types=[
        pltpu.VMEM((MAX_VISITS * EMBED_DIM,), jnp.float32),
        pltpu.VMEM((B_PER_W,), jnp.int32),
        pltpu.VMEM((B_PER_W,), jnp.int32),
        pltpu.VMEM((EMBED_DIM, B_PER_W), jnp.float32),
        pltpu.VMEM((EMBED_DIM, B_PER_W), jnp.float32),
        pltpu.SemaphoreType.DMA,
        pltpu.SemaphoreType.DMA,
        pltpu.SemaphoreType.DMA,
        pltpu.SemaphoreType.DMA,
    ],
    compiler_params=pltpu.CompilerParams(needs_layout_passes=False),
)
def _embed_kernel(ids_t_hbm, tab_t_hbm, out_hbm, tab_v, ids0, ids1,
                  out0, out1, isem0, isem1, osem0, osem1):
    wid = lax.axis_index("s") * NC + lax.axis_index("c")
    b0 = wid * B_PER_W

    ids_bufs = (ids0, ids1)
    out_bufs = (out0, out1)
    isems = (isem0, isem1)
    osems = (osem0, osem1)

    pltpu.sync_copy(tab_t_hbm, tab_v)

    pltpu.async_copy(ids_t_hbm.at[0, pl.ds(b0, B_PER_W)], ids0, isem0)
    pltpu.async_copy(ids_t_hbm.at[1, pl.ds(b0, B_PER_W)], ids1, isem1)

    def step(i, _):
        for half in (0, 1):
            s = 2 * i + half
            ids_v = ids_bufs[half]
            out_v = out_bufs[half]
            pltpu.make_async_copy(
                ids_t_hbm.at[0, pl.ds(b0, B_PER_W)], ids_v, isems[half]
            ).wait()
            @pl.when(s >= 2)
            def _():
                pltpu.make_async_copy(
                    out_v, out_hbm.at[0, :, pl.ds(b0, B_PER_W)], osems[half]
                ).wait()

            def gather_group(j, _):
                idvec = ids_v[pl.ds(j * 16, 16)]
                for e in range(EMBED_DIM):
                    out_v[e, pl.ds(j * 16, 16)] = plsc.load_gather(
                        tab_v, [idvec + (e * MAX_VISITS)]
                    )
                return ()

            lax.fori_loop(0, NJ, gather_group, ())

            @pl.when(s + 2 < SEQ)
            def _():
                pltpu.async_copy(
                    ids_t_hbm.at[s + 2, pl.ds(b0, B_PER_W)], ids_v, isems[half]
                )

            pltpu.async_copy(
                out_v, out_hbm.at[s, :, pl.ds(b0, B_PER_W)], osems[half]
            )
        return ()

    lax.fori_loop(0, SEQ // 2, step, ())

    for half in (0, 1):
        pltpu.make_async_copy(
            out_bufs[half], out_hbm.at[0, :, pl.ds(b0, B_PER_W)], osems[half]
        ).wait()


def kernel(visit_ids, table):
    ids_t = visit_ids.T.astype(jnp.int32)
    tab_t = table.T.reshape(EMBED_DIM * MAX_VISITS)
    out = _embed_kernel(ids_t, tab_t)
    return out.transpose(2, 0, 1)


# speedup vs baseline: 26.2194x; 3.4862x over previous
import functools

import jax
import jax.numpy as jnp
from jax import lax
from jax.experimental import pallas as pl
from jax.experimental.pallas import tpu as pltpu
from jax.experimental.pallas import tpu_sc as plsc

MAX_VISITS = 512
EMBED_DIM = 64
BATCH = 16384
SEQ = 200

_info = plsc.get_sparse_core_info()
NC, NS = _info.num_cores, _info.num_subcores
NW = NC * NS
B_PER_W = BATCH // NW
NJ = B_PER_W // 16

_mesh = plsc.VectorSubcoreMesh(core_axis_name="c", subcore_axis_name="s")


@functools.partial(
    pl.kernel,
    mesh=_mesh,
    out_type=jax.ShapeDtypeStruct((SEQ, EMBED_DIM, BATCH), jnp.float32),
    scratch_types=[
        pltpu.VMEM((MAX_VISITS * EMBED_DIM,), jnp.float32),
        pltpu.VMEM((B_PER_W,), jnp.int32),
        pltpu.VMEM((B_PER_W,), jnp.int32),
        pltpu.VMEM((EMBED_DIM, B_PER_W), jnp.float32),
        pltpu.VMEM((EMBED_DIM, B_PER_W), jnp.float32),
        pltpu.SemaphoreType.DMA,
        pltpu.SemaphoreType.DMA,
        pltpu.SemaphoreType.DMA,
        pltpu.SemaphoreType.DMA,
    ],
    compiler_params=pltpu.CompilerParams(needs_layout_passes=False),
)
def _embed_kernel(ids_t_hbm, tab_t_hbm, out_hbm, tab_v, ids0, ids1,
                  out0, out1, isem0, isem1, osem0, osem1):
    wid = lax.axis_index("s") * NC + lax.axis_index("c")
    b0 = wid * B_PER_W

    ids_bufs = (ids0, ids1)
    out_bufs = (out0, out1)
    isems = (isem0, isem1)
    osems = (osem0, osem1)

    pltpu.sync_copy(tab_t_hbm, tab_v)

    pltpu.async_copy(ids_t_hbm.at[0, pl.ds(b0, B_PER_W)], ids0, isem0)
    pltpu.async_copy(ids_t_hbm.at[1, pl.ds(b0, B_PER_W)], ids1, isem1)

    def step(i, _):
        for half in (0, 1):
            s = 2 * i + half
            ids_v = ids_bufs[half]
            out_v = out_bufs[half]
            pltpu.make_async_copy(
                ids_t_hbm.at[0, pl.ds(b0, B_PER_W)], ids_v, isems[half]
            ).wait()
            @pl.when(s >= 2)
            def _():
                pltpu.make_async_copy(
                    out_v, out_hbm.at[0, :, pl.ds(b0, B_PER_W)], osems[half]
                ).wait()

            @plsc.parallel_loop(0, NJ, step=1, unroll=2)
            def _gather_group(j):
                idvec = ids_v[pl.ds(j * 16, 16)]
                for e in range(EMBED_DIM):
                    out_v[e, pl.ds(j * 16, 16)] = plsc.load_gather(
                        tab_v, [idvec + (e * MAX_VISITS)]
                    )

            @pl.when(s + 2 < SEQ)
            def _():
                pltpu.async_copy(
                    ids_t_hbm.at[s + 2, pl.ds(b0, B_PER_W)], ids_v, isems[half]
                )

            pltpu.async_copy(
                out_v, out_hbm.at[s, :, pl.ds(b0, B_PER_W)], osems[half]
            )
        return ()

    lax.fori_loop(0, SEQ // 2, step, ())

    for half in (0, 1):
        pltpu.make_async_copy(
            out_bufs[half], out_hbm.at[0, :, pl.ds(b0, B_PER_W)], osems[half]
        ).wait()


def kernel(visit_ids, table):
    ids_t = visit_ids.T.astype(jnp.int32)
    tab_t = table.T.reshape(EMBED_DIM * MAX_VISITS)
    out = _embed_kernel(ids_t, tab_t)
    return out.transpose(2, 0, 1)
